# baseline (speedup 1.0000x reference)
import functools

import jax
import jax.numpy as jnp
from jax import lax
from jax.experimental import pallas as pl
from jax.experimental.pallas import tpu as pltpu
from jax.experimental.pallas import tpu_sc as plsc

_VOCAB = 1000
_R = 32
_NBUF = 2
_NW = 32
_ROWS = 4096 * 20
_RPW = _ROWS // _NW
_NCHUNK = _RPW // _R
_NSUPER = _NCHUNK // _NBUF


def _sc_body(x_hbm, out_hbm, xall, buf0, buf1, sem0, sem1):
    c = lax.axis_index("c")
    s = lax.axis_index("s")
    wid = s * 2 + c
    base_row = wid * _RPW

    pltpu.sync_copy(x_hbm.at[pl.ds(base_row, _RPW)], xall)

    zeros16 = jnp.zeros((16,), jnp.float32)
    ones16 = jnp.ones((16,), jnp.float32)
    lane = lax.iota(jnp.int32, 16)

    def zbody(i, carry):
        buf0[pl.ds(i * 16, 16)] = zeros16
        buf1[pl.ds(i * 16, 16)] = zeros16
        return carry

    lax.fori_loop(0, _R * _VOCAB // 16, zbody, 0)

    bufs = (buf0, buf1)
    sems = (sem0, sem1)

    def scatter(buf, chunk, val16):
        for j in range(_R // 16):
            xv = xall[pl.ds(chunk * _R + j * 16, 16)]
            pos = xv + (lane + j * 16) * _VOCAB
            plsc.store_scatter(buf, [pos], val16)

    def dma(buf, chunk, sem):
        dst0 = (base_row + chunk * _R) * _VOCAB
        return pltpu.make_async_copy(
            buf, out_hbm.at[pl.ds(dst0, _R * _VOCAB)], sem
        )

    for j in range(_NBUF):
        scatter(bufs[j], j, ones16)
        dma(bufs[j], j, sems[j]).start()

    def body(ss, carry):
        for j in range(_NBUF):
            chunk = ss * _NBUF + j
            dma(bufs[j], chunk, sems[j]).wait()
            scatter(bufs[j], chunk - _NBUF, zeros16)
            scatter(bufs[j], chunk, ones16)
            dma(bufs[j], chunk, sems[j]).start()
        return carry

    lax.fori_loop(1, _NSUPER, body, 0)

    for j in range(_NBUF):
        dma(bufs[j], 0, sems[j]).wait()


@functools.partial(
    pl.kernel,
    mesh=plsc.VectorSubcoreMesh(core_axis_name="c", subcore_axis_name="s"),
    compiler_params=pltpu.CompilerParams(needs_layout_passes=False),
    out_type=jax.ShapeDtypeStruct((_ROWS * _VOCAB,), jnp.float32),
    scratch_types=[
        pltpu.VMEM((_RPW,), jnp.int32),
        pltpu.VMEM((_R * _VOCAB,), jnp.float32),
        pltpu.VMEM((_R * _VOCAB,), jnp.float32),
        pltpu.SemaphoreType.DMA,
        pltpu.SemaphoreType.DMA,
    ],
)
def _sc_onehot(x_hbm, out_hbm, xall, buf0, buf1, sem0, sem1):
    _sc_body(x_hbm, out_hbm, xall, buf0, buf1, sem0, sem1)


def kernel(x, W):
    B, H = x.shape
    xf = x.reshape(B * H).astype(jnp.int32)
    out = _sc_onehot(xf)
    return out.reshape(B, H, _VOCAB)


# speedup vs baseline: 1.2774x; 1.2774x over previous
import jax
import jax.numpy as jnp
from jax.experimental import pallas as pl

_VOCAB = 1000
_RB = 256
_VB = 128


def _onehot_block(x_ref, o_ref):
    j = pl.program_id(1)
    idx = x_ref[...]
    iota = jax.lax.broadcasted_iota(jnp.int32, o_ref.shape, 2) + j * _VB
    o_ref[...] = (idx == iota).astype(jnp.float32)


def kernel(x, W):
    B, H = x.shape
    x3 = x.reshape(B, H, 1).astype(jnp.int32)
    out = pl.pallas_call(
        _onehot_block,
        grid=(B // _RB, _VOCAB // _VB + 1),
        in_specs=[pl.BlockSpec((_RB, H, 1), lambda i, j: (i, 0, 0))],
        out_specs=pl.BlockSpec((_RB, H, _VB), lambda i, j: (i, 0, j)),
        out_shape=jax.ShapeDtypeStruct((B, H, _VOCAB), jnp.float32),
    )(x3)
    return out


# speedup vs baseline: 1.4833x; 1.1612x over previous
import functools

import jax
import jax.numpy as jnp
from jax import lax
from jax.experimental import pallas as pl
from jax.experimental.pallas import tpu as pltpu
from jax.experimental.pallas import tpu_sc as plsc

_VOCAB = 1000
_B = 4096
_H = 20
_NW = 32
_BPW = _B // _NW
_CB = 2
_NCHUNK = _BPW // _CB
_NBUF = 2
_NSUPER = _NCHUNK // _NBUF
_TPB = 24 * 1024
_CHW = _CB * _TPB


def _sc_body(x_hbm, out_hbm, xall, buf0, buf1, sem0, sem1):
    c = lax.axis_index("c")
    s = lax.axis_index("s")
    wid = s * 2 + c
    base_b = wid * _BPW

    pltpu.sync_copy(x_hbm.at[pl.ds(base_b * _H, _BPW * _H)], xall)

    zeros16 = jnp.zeros((16,), jnp.float32)
    ones16 = jnp.ones((16,), jnp.float32)
    lane = lax.iota(jnp.int32, 16)

    def zbody(i, carry):
        for b in range(_CB):
            for h in range(_H):
                buf0[b, h, pl.ds(i * 16, 16)] = zeros16
                buf1[b, h, pl.ds(i * 16, 16)] = zeros16
        return carry

    lax.fori_loop(0, _VOCAB // 16, zbody, 0)
    for b in range(_CB):
        for h in range(_H):
            buf0[b, h, pl.ds(_VOCAB - 16, 16)] = zeros16
            buf1[b, h, pl.ds(_VOCAB - 16, 16)] = zeros16

    bufs = (buf0, buf1)
    sems = (sem0, sem1)

    def scatter(buf, chunk, val16):
        p0_base = chunk * _CB * _H
        for g in range(3):
            p = lane + g * 16
            xv = xall[pl.ds(p0_base + g * 16, 16)]
            b_local = p // _H
            h = p - b_local * _H
            if g < 2:
                plsc.store_scatter(buf, [b_local, h, xv], val16)
            else:
                plsc.store_scatter(
                    buf, [b_local, h, xv], val16, mask=p < _CB * _H
                )

    def dma(buf, chunk, sem):
        b0 = base_b + chunk * _CB
        return pltpu.make_async_copy(
            buf, out_hbm.at[pl.ds(b0, _CB)], sem
        )

    for j in range(_NBUF):
        scatter(bufs[j], j, ones16)
        dma(bufs[j], j, sems[j]).start()

    def body(ss, carry):
        for j in range(_NBUF):
            chunk = ss * _NBUF + j
            dma(bufs[j], chunk, sems[j]).wait()
            scatter(bufs[j], chunk - _NBUF, zeros16)
            scatter(bufs[j], chunk, ones16)
            dma(bufs[j], chunk, sems[j]).start()
        return carry

    lax.fori_loop(1, _NSUPER, body, 0)

    for j in range(_NBUF):
        dma(bufs[j], 0, sems[j]).wait()


@functools.partial(
    pl.kernel,
    mesh=plsc.VectorSubcoreMesh(core_axis_name="c", subcore_axis_name="s"),
    compiler_params=pltpu.CompilerParams(
        needs_layout_passes=False, use_tc_tiling_on_sc=True
    ),
    out_type=jax.ShapeDtypeStruct((_B, _H, _VOCAB), jnp.float32),
    scratch_types=[
        pltpu.VMEM((_BPW * _H,), jnp.int32),
        pltpu.VMEM((_CB, _H, _VOCAB), jnp.float32),
        pltpu.VMEM((_CB, _H, _VOCAB), jnp.float32),
        pltpu.SemaphoreType.DMA,
        pltpu.SemaphoreType.DMA,
    ],
)
def _sc_onehot(x_hbm, out_hbm, xall, buf0, buf1, sem0, sem1):
    _sc_body(x_hbm, out_hbm, xall, buf0, buf1, sem0, sem1)


def kernel(x, W):
    B, H = x.shape
    xf = x.reshape(B * H).astype(jnp.int32)
    return _sc_onehot(xf)


# speedup vs baseline: 1.5058x; 1.0151x over previous
import functools

import jax
import jax.numpy as jnp
from jax import lax
from jax.experimental import pallas as pl
from jax.experimental.pallas import tpu as pltpu
from jax.experimental.pallas import tpu_sc as plsc

_VOCAB = 1000
_B = 4096
_H = 20
_NW = 32
_BPW = _B // _NW
_CB = 4
_NCHUNK = _BPW // _CB
_NPAIR = _CB * _H


def _sc_body(x_hbm, out_hbm, xall, buf, sem):
    c = lax.axis_index("c")
    s = lax.axis_index("s")
    wid = s * 2 + c
    base_b = wid * _BPW

    pltpu.sync_copy(x_hbm.at[pl.ds(base_b * _H, _BPW * _H)], xall)

    zeros16 = jnp.zeros((16,), jnp.float32)
    ones16 = jnp.ones((16,), jnp.float32)
    lane = lax.iota(jnp.int32, 16)

    def zbody(i, carry):
        for b in range(_CB):
            for h in range(_H):
                buf[b, h, pl.ds(i * 16, 16)] = zeros16
        return carry

    lax.fori_loop(0, _VOCAB // 16, zbody, 0)
    for b in range(_CB):
        for h in range(_H):
            buf[b, h, pl.ds(_VOCAB - 16, 16)] = zeros16

    def scatter(chunk, val16):
        p0_base = chunk * _NPAIR
        for g in range(_NPAIR // 16):
            p = lane + g * 16
            xv = xall[pl.ds(p0_base + g * 16, 16)]
            b_local = p // _H
            h = p - b_local * _H
            plsc.store_scatter(buf, [b_local, h, xv], val16)

    def dma(chunk):
        b0 = base_b + chunk * _CB
        return pltpu.make_async_copy(buf, out_hbm.at[pl.ds(b0, _CB)], sem)

    scatter(0, ones16)
    dma(0).start()

    def body(chunk, carry):
        dma(chunk).wait()
        scatter(chunk - 1, zeros16)
        scatter(chunk, ones16)
        dma(chunk).start()
        return carry

    lax.fori_loop(1, _NCHUNK, body, 0)
    dma(0).wait()


@functools.partial(
    pl.kernel,
    mesh=plsc.VectorSubcoreMesh(core_axis_name="c", subcore_axis_name="s"),
    compiler_params=pltpu.CompilerParams(
        needs_layout_passes=False, use_tc_tiling_on_sc=True
    ),
    out_type=jax.ShapeDtypeStruct((_B, _H, _VOCAB), jnp.float32),
    scratch_types=[
        pltpu.VMEM((_BPW * _H,), jnp.int32),
        pltpu.VMEM((_CB, _H, _VOCAB), jnp.float32),
        pltpu.SemaphoreType.DMA,
    ],
)
def _sc_onehot(x_hbm, out_hbm, xall, buf, sem):
    _sc_body(x_hbm, out_hbm, xall, buf, sem)


def kernel(x, W):
    B, H = x.shape
    xf = x.reshape(B * H).astype(jnp.int32)
    return _sc_onehot(xf)
